# Initial kernel scaffold; baseline (speedup 1.0000x reference)
import jax
import jax.numpy as jnp
from jax.experimental import pallas as pl


def kernel(x0, edge_index0, edge_weight0, x1, edge_index1, edge_weight1, W1_0, b1_0, a_0, W2_0, b2_0, W1_1, b1_1, a_1, W2_1, b2_1, Wd):
    raise NotImplementedError("write your pallas kernel here")


# speedup vs baseline: 4.4224x; 4.4224x over previous
import functools

import jax
import jax.numpy as jnp
from jax import lax
from jax.experimental import pallas as pl
from jax.experimental.pallas import tpu as pltpu, tpu_sc as plsc

N = 10000
NP = 10240
E = 320000
F = 128
L = 16
NC = 2
NS = 16
EPT = E // NS
CK = 80
NCHUNK = EPT // CK
RPT = NP // NS
BR = 1024
GR = NP // BR

_f32 = jnp.float32


def _sc_mesh():
  return plsc.VectorSubcoreMesh(
      core_axis_name="c", subcore_axis_name="s", num_cores=NC, num_subcores=NS)


def _sc_spmm(hsf, src, dst, ew):
  @functools.partial(
      pl.kernel,
      mesh=_sc_mesh(),
      out_type=jax.ShapeDtypeStruct((NC * NP, F), _f32),
      scratch_types=[
          pltpu.VMEM((CK,), jnp.int32),
          pltpu.VMEM((CK,), jnp.int32),
          pltpu.VMEM((CK,), _f32),
          pltpu.VMEM((CK, F), _f32),
          pltpu.VMEM_SHARED((NP, F), _f32),
          pltpu.SemaphoreType.DMA,
      ],
  )
  def k(hs_hbm, src_hbm, dst_hbm, ew_hbm, out_hbm, sidx, didx, ewv, rows, acc,
        sem):
    c = lax.axis_index("c")
    s = lax.axis_index("s")

    def zrow(i, carry):
      for j in range(F // L):
        rows[i, pl.ds(j * L, L)] = jnp.zeros((L,), _f32)
      return carry
    lax.fori_loop(0, CK, zrow, 0)
    base_r = s * RPT
    for b in range(RPT // CK):
      pltpu.sync_copy(rows, acc.at[pl.ds(base_r + b * CK, CK)])
    plsc.subcore_barrier()

    e0 = s * EPT
    coff = c * NP

    def chunk(i, carry):
      eb = e0 + i * CK
      pltpu.sync_copy(src_hbm.at[pl.ds(eb, CK)], sidx)
      pltpu.sync_copy(dst_hbm.at[pl.ds(eb, CK)], didx)
      pltpu.sync_copy(ew_hbm.at[pl.ds(eb, CK)], ewv)
      for j in range(CK // L):
        sidx[pl.ds(j * L, L)] = sidx[pl.ds(j * L, L)] + coff
      pltpu.async_copy(hs_hbm.at[sidx], rows, sem).wait()

      def scale(g, carry2):
        wvec = ewv[pl.ds(g * L, L)]
        for t in range(L):
          w = wvec[t]
          row = g * L + t
          for j in range(F // L):
            rows[row, pl.ds(j * L, L)] = rows[row, pl.ds(j * L, L)] * w
        return carry2
      lax.fori_loop(0, CK // L, scale, 0)
      pltpu.sync_copy(rows, acc.at[didx], add=True)
      return carry
    lax.fori_loop(0, NCHUNK, chunk, 0)

    plsc.subcore_barrier()
    pltpu.sync_copy(acc.at[pl.ds(base_r, RPT)],
                    out_hbm.at[pl.ds(coff + base_r, RPT)])

  return k(hsf, src, dst, ew)


def _sc_deg_shuffle(dstf, ewf, x1p, permp):
  @functools.partial(
      pl.kernel,
      mesh=_sc_mesh(),
      out_type=(
          jax.ShapeDtypeStruct((NC * NP, F), _f32),
          jax.ShapeDtypeStruct((NP, F), _f32),
      ),
      scratch_types=[
          pltpu.VMEM((CK,), jnp.int32),
          pltpu.VMEM((CK,), _f32),
          pltpu.VMEM((CK, F), _f32),
          pltpu.VMEM_SHARED((NP, F), _f32),
          pltpu.SemaphoreType.DMA,
      ],
  )
  def k(dst_hbm, ew_hbm, x1_hbm, perm_hbm, deg_hbm, xs_hbm,
        didx, ewv, rows, acc, sem):
    c = lax.axis_index("c")
    s = lax.axis_index("s")

    def zrow(i, carry):
      for j in range(F // L):
        rows[i, pl.ds(j * L, L)] = jnp.zeros((L,), _f32)
      return carry
    lax.fori_loop(0, CK, zrow, 0)
    base_r = s * RPT
    for b in range(RPT // CK):
      pltpu.sync_copy(rows, acc.at[pl.ds(base_r + b * CK, CK)])
    plsc.subcore_barrier()

    e0 = c * E + s * EPT

    def chunk(i, carry):
      eb = e0 + i * CK
      pltpu.sync_copy(dst_hbm.at[pl.ds(eb, CK)], didx)
      pltpu.sync_copy(ew_hbm.at[pl.ds(eb, CK)], ewv)

      def splat(g, carry2):
        wvec = ewv[pl.ds(g * L, L)]
        for t in range(L):
          w = jnp.full((L,), wvec[t], _f32)
          for j in range(F // L):
            rows[g * L + t, pl.ds(j * L, L)] = w
        return carry2
      lax.fori_loop(0, CK // L, splat, 0)
      pltpu.sync_copy(rows, acc.at[didx], add=True)
      return carry
    lax.fori_loop(0, NCHUNK, chunk, 0)
    plsc.subcore_barrier()
    pltpu.sync_copy(acc.at[pl.ds(base_r, RPT)],
                    deg_hbm.at[pl.ds(c * NP + base_r, RPT)])

    base_x = (s * NC + c) * (NP // (NC * NS))
    for q in range(NP // (NC * NS) // CK):
      pltpu.sync_copy(perm_hbm.at[pl.ds(base_x + q * CK, CK)], didx)
      pltpu.async_copy(x1_hbm.at[didx], rows, sem).wait()
      pltpu.sync_copy(rows, xs_hbm.at[pl.ds(base_x + q * CK, CK)])

  return k(dstf, ewf, x1p, permp)


def _tc_layer_in(xin, w1s, deg2):
  def body(x_ref, w_ref, d_ref, o_ref):
    dinv = lax.rsqrt(d_ref[0] + 1.0)
    h = jnp.dot(x_ref[0, 0], w_ref[0], preferred_element_type=_f32)
    o_ref[0, 0] = dinv * h

  return pl.pallas_call(
      body,
      grid=(2, 2, GR),
      in_specs=[
          pl.BlockSpec((1, 1, BR, F), lambda v, p, r: (v, p, r, 0)),
          pl.BlockSpec((1, F, F), lambda v, p, r: (v, 0, 0)),
          pl.BlockSpec((1, BR, F), lambda v, p, r: (v, r, 0)),
      ],
      out_specs=pl.BlockSpec((1, 1, BR, F), lambda v, p, r: (v, p, r, 0)),
      out_shape=jax.ShapeDtypeStruct((2, 2, NP, F), _f32),
  )(xin, w1s, deg2)


def _tc_mid(s1, h1, deg2, b1s, a_s, w2s):
  def body(s_ref, h_ref, d_ref, b_ref, a_ref, w_ref, o_ref):
    v = pl.program_id(0)
    dinv = lax.rsqrt(d_ref[0] + 1.0)
    b = jnp.where(v == 0, b_ref[0], b_ref[1])
    a = jnp.where(v == 0, a_ref[0], a_ref[1])
    z = dinv * (s_ref[0, 0] + h_ref[0, 0]) + b
    z = jnp.where(z > 0, z, a * z)
    o_ref[0, 0] = dinv * jnp.dot(z, w_ref[0], preferred_element_type=_f32)

  return pl.pallas_call(
      body,
      grid=(2, 2, GR),
      in_specs=[
          pl.BlockSpec((1, 1, BR, F), lambda v, p, r: (v, p, r, 0)),
          pl.BlockSpec((1, 1, BR, F), lambda v, p, r: (v, p, r, 0)),
          pl.BlockSpec((1, BR, F), lambda v, p, r: (v, r, 0)),
          pl.BlockSpec((2, F), lambda v, p, r: (0, 0)),
          pl.BlockSpec((2, F), lambda v, p, r: (0, 0)),
          pl.BlockSpec((1, F, F), lambda v, p, r: (v, 0, 0)),
      ],
      out_specs=pl.BlockSpec((1, 1, BR, F), lambda v, p, r: (v, p, r, 0)),
      out_shape=jax.ShapeDtypeStruct((2, 2, NP, F), _f32),
  )(s1, h1, deg2, b1s, a_s, w2s)


def _tc_out(s2, h2, deg2, b2s):
  def body(s_ref, h_ref, d_ref, b_ref, o_ref, c_ref):
    v = pl.program_id(0)
    p = pl.program_id(1)
    r = pl.program_id(2)
    dinv = lax.rsqrt(d_ref[0] + 1.0)
    b = jnp.where(v == 0, b_ref[0], b_ref[1])
    o = dinv * (s_ref[0, 0] + h_ref[0, 0]) + b
    o_ref[0, 0] = o

    @pl.when(jnp.logical_and(v == 0, jnp.logical_and(p == 0, r == 0)))
    def _():
      c_ref[...] = jnp.zeros((2, F), _f32)

    @pl.when(p == 0)
    def _():
      rows = r * BR + lax.broadcasted_iota(jnp.int32, (BR, F), 0)
      contrib = jnp.sum(jnp.where(rows < N, o, 0.0), axis=0)

      @pl.when(v == 0)
      def _():
        c_ref[0, :] += contrib

      @pl.when(v == 1)
      def _():
        c_ref[1, :] += contrib

  return pl.pallas_call(
      body,
      grid=(2, 2, GR),
      in_specs=[
          pl.BlockSpec((1, 1, BR, F), lambda v, p, r: (v, p, r, 0)),
          pl.BlockSpec((1, 1, BR, F), lambda v, p, r: (v, p, r, 0)),
          pl.BlockSpec((1, BR, F), lambda v, p, r: (v, r, 0)),
          pl.BlockSpec((2, F), lambda v, p, r: (0, 0)),
      ],
      out_specs=[
          pl.BlockSpec((1, 1, BR, F), lambda v, p, r: (v, p, r, 0)),
          pl.BlockSpec((2, F), lambda v, p, r: (0, 0)),
      ],
      out_shape=[
          jax.ShapeDtypeStruct((2, 2, NP, F), _f32),
          jax.ShapeDtypeStruct((2, F), _f32),
      ],
  )(s2, h2, deg2, b2s)


def _tc_logits(outv, csum, wd):
  def body(o_ref, c_ref, w_ref, l_ref):
    v = pl.program_id(2)
    cv = jnp.where(v == 0, c_ref[0], c_ref[1])
    cvec = jax.nn.sigmoid(cv * (1.0 / N))
    wc = jnp.sum(w_ref[...] * cvec[None, :], axis=1)
    sc = jnp.sum(o_ref[0, 0] * wc[None, :], axis=1)

    @pl.when(v == 0)
    def _():
      l_ref[0, :] = sc

    @pl.when(v == 1)
    def _():
      l_ref[1, :] = sc

  return pl.pallas_call(
      body,
      grid=(2, GR, 2),
      in_specs=[
          pl.BlockSpec((1, 1, BR, F), lambda p, r, v: (v, p, r, 0)),
          pl.BlockSpec((2, F), lambda p, r, v: (0, 0)),
          pl.BlockSpec((F, F), lambda p, r, v: (0, 0)),
      ],
      out_specs=pl.BlockSpec((2, BR), lambda p, r, v: (0, p * GR + r)),
      out_shape=jax.ShapeDtypeStruct((2, 2 * NP), _f32),
  )(outv, csum, wd)


def kernel(x0, edge_index0, edge_weight0, x1, edge_index1, edge_weight1,
           W1_0, b1_0, a_0, W2_0, b2_0, W1_1, b1_1, a_1, W2_1, b2_1, Wd):
  x0p = jnp.pad(x0, ((0, NP - N), (0, 0)))
  x1p = jnp.pad(x1, ((0, NP - N), (0, 0)))
  perm = jax.random.permutation(jax.random.key(42), N).astype(jnp.int32)
  permp = jnp.pad(perm, (0, NP - N))

  dstf = jnp.concatenate([edge_index0[1], edge_index1[1]])
  ewf = jnp.concatenate([edge_weight0, edge_weight1])
  degf, xsp = _sc_deg_shuffle(dstf, ewf, x1p, permp)
  deg2 = degf.reshape(NC, NP, F)

  xin = jnp.stack([jnp.stack([x0p, xsp]), jnp.stack([x1p, xsp])])
  w1s = jnp.stack([W1_0, W1_1])
  w2s = jnp.stack([W2_0, W2_1])
  b1s = jnp.stack([b1_0, b1_1])
  b2s = jnp.stack([b2_0, b2_1])
  a_s = jnp.stack([jnp.broadcast_to(a_0, (F,)), jnp.broadcast_to(a_1, (F,))])

  h1 = _tc_layer_in(xin, w1s, deg2)

  s1_list = []
  for v in range(2):
    src, dst, ew = (edge_index0[0], edge_index0[1], edge_weight0) if v == 0 \
        else (edge_index1[0], edge_index1[1], edge_weight1)
    s1_list.append(_sc_spmm(h1[v].reshape(NC * NP, F), src, dst, ew))
  s1 = jnp.stack([s.reshape(NC, NP, F) for s in s1_list])

  h2 = _tc_mid(s1, h1, deg2, b1s, a_s, w2s)

  s2_list = []
  for v in range(2):
    src, dst, ew = (edge_index0[0], edge_index0[1], edge_weight0) if v == 0 \
        else (edge_index1[0], edge_index1[1], edge_weight1)
    s2_list.append(_sc_spmm(h2[v].reshape(NC * NP, F), src, dst, ew))
  s2 = jnp.stack([s.reshape(NC, NP, F) for s in s2_list])

  outv, csum = _tc_out(s2, h2, deg2, b2s)
  logits_p = _tc_logits(outv, csum, Wd)
  return jnp.concatenate([logits_p[:, :N], logits_p[:, NP:NP + N]], axis=1)


# speedup vs baseline: 8.2225x; 1.8593x over previous
import functools

import jax
import jax.numpy as jnp
from jax import lax
from jax.experimental import pallas as pl
from jax.experimental.pallas import tpu as pltpu, tpu_sc as plsc

N = 10000
NP = 10240
E = 320000
F = 128
L = 16
NC = 2
NS = 16
EPT = E // NS
CK = 80
NCHUNK = EPT // CK
RPT = NP // NS
BR = 1024
GR = NP // BR

_f32 = jnp.float32


def _sc_mesh():
  return plsc.VectorSubcoreMesh(
      core_axis_name="c", subcore_axis_name="s", num_cores=NC, num_subcores=NS)


def _sc_spmm(hsf, src, dst, ew):
  @functools.partial(
      pl.kernel,
      mesh=_sc_mesh(),
      out_type=jax.ShapeDtypeStruct((NC * NP, F), _f32),
      scratch_types=[
          pltpu.VMEM((2, CK), jnp.int32),
          pltpu.VMEM((2, CK), jnp.int32),
          pltpu.VMEM((2, CK), _f32),
          pltpu.VMEM((CK, F), _f32),
          pltpu.VMEM((CK, F), _f32),
          pltpu.VMEM_SHARED((NP, F), _f32),
          pltpu.SemaphoreType.DMA,
          pltpu.SemaphoreType.DMA,
          pltpu.SemaphoreType.DMA,
          pltpu.SemaphoreType.DMA,
      ],
  )
  def k(hs_hbm, src_hbm, dst_hbm, ew_hbm, out_hbm, sidx, didx, ewv,
        rows0, rows1, acc, isem0, isem1, gsem0, gsem1):
    c = lax.axis_index("c")
    s = lax.axis_index("s")
    rows_b = (rows0, rows1)
    isem_b = (isem0, isem1)
    gsem_b = (gsem0, gsem1)

    def zrow(i, carry):
      for j in range(F // L):
        rows0[i, pl.ds(j * L, L)] = jnp.zeros((L,), _f32)
      return carry
    lax.fori_loop(0, CK, zrow, 0)
    base_r = s * RPT
    for b in range(RPT // CK):
      pltpu.sync_copy(rows0, acc.at[pl.ds(base_r + b * CK, CK)])
    plsc.subcore_barrier()

    e0 = s * EPT
    coff = c * NP

    def issue_idx(i, p):
      eb = e0 + i * CK
      a1 = pltpu.async_copy(src_hbm.at[pl.ds(eb, CK)], sidx.at[p], isem_b[p])
      a2 = pltpu.async_copy(dst_hbm.at[pl.ds(eb, CK)], didx.at[p], isem_b[p])
      a3 = pltpu.async_copy(ew_hbm.at[pl.ds(eb, CK)], ewv.at[p], isem_b[p])
      return a1, a2, a3

    def issue_gather(p, descs):
      for d in descs:
        d.wait()
      for j in range(CK // L):
        sidx[p, pl.ds(j * L, L)] = sidx[p, pl.ds(j * L, L)] + coff
      return pltpu.async_copy(hs_hbm.at[sidx.at[p]], rows_b[p], gsem_b[p])

    def process(p, gd):
      gd.wait()
      rows = rows_b[p]

      def scale(g, carry2):
        wvec = ewv[p, pl.ds(g * L, L)]
        for t in range(L):
          w = wvec[t]
          row = g * L + t
          for j in range(F // L):
            rows[row, pl.ds(j * L, L)] = rows[row, pl.ds(j * L, L)] * w
        return carry2
      lax.fori_loop(0, CK // L, scale, 0)
      pltpu.sync_copy(rows, acc.at[didx.at[p]], add=True)

    issue_gather(0, issue_idx(0, 0))

    def pair2(i2, carry):
      i = i2 * 2
      g1 = issue_gather(1, issue_idx(i + 1, 1))
      gd0 = pltpu.make_async_copy(hs_hbm.at[sidx.at[0]], rows0, gsem0)
      process(0, gd0)

      @pl.when(i + 2 < NCHUNK)
      def _():
        issue_gather(0, issue_idx(i + 2, 0))
      process(1, g1)
      return carry
    lax.fori_loop(0, NCHUNK // 2, pair2, 0)

    plsc.subcore_barrier()
    pltpu.sync_copy(acc.at[pl.ds(base_r, RPT)],
                    out_hbm.at[pl.ds(coff + base_r, RPT)])

  return k(hsf, src, dst, ew)


def _sc_deg_shuffle(dstf, ewf, x1p, permp):
  @functools.partial(
      pl.kernel,
      mesh=_sc_mesh(),
      out_type=(
          jax.ShapeDtypeStruct((NC * NP, F), _f32),
          jax.ShapeDtypeStruct((NP, F), _f32),
      ),
      scratch_types=[
          pltpu.VMEM((CK,), jnp.int32),
          pltpu.VMEM((CK,), _f32),
          pltpu.VMEM((CK, F), _f32),
          pltpu.VMEM_SHARED((NP, F), _f32),
          pltpu.SemaphoreType.DMA,
      ],
  )
  def k(dst_hbm, ew_hbm, x1_hbm, perm_hbm, deg_hbm, xs_hbm,
        didx, ewv, rows, acc, sem):
    c = lax.axis_index("c")
    s = lax.axis_index("s")

    def zrow(i, carry):
      for j in range(F // L):
        rows[i, pl.ds(j * L, L)] = jnp.zeros((L,), _f32)
      return carry
    lax.fori_loop(0, CK, zrow, 0)
    base_r = s * RPT
    for b in range(RPT // CK):
      pltpu.sync_copy(rows, acc.at[pl.ds(base_r + b * CK, CK)])
    plsc.subcore_barrier()

    e0 = c * E + s * EPT

    def chunk(i, carry):
      eb = e0 + i * CK
      pltpu.sync_copy(dst_hbm.at[pl.ds(eb, CK)], didx)
      pltpu.sync_copy(ew_hbm.at[pl.ds(eb, CK)], ewv)

      def splat(g, carry2):
        wvec = ewv[pl.ds(g * L, L)]
        for t in range(L):
          w = jnp.full((L,), wvec[t], _f32)
          for j in range(F // L):
            rows[g * L + t, pl.ds(j * L, L)] = w
        return carry2
      lax.fori_loop(0, CK // L, splat, 0)
      pltpu.sync_copy(rows, acc.at[didx], add=True)
      return carry
    lax.fori_loop(0, NCHUNK, chunk, 0)
    plsc.subcore_barrier()
    pltpu.sync_copy(acc.at[pl.ds(base_r, RPT)],
                    deg_hbm.at[pl.ds(c * NP + base_r, RPT)])

    base_x = (s * NC + c) * (NP // (NC * NS))
    for q in range(NP // (NC * NS) // CK):
      pltpu.sync_copy(perm_hbm.at[pl.ds(base_x + q * CK, CK)], didx)
      pltpu.async_copy(x1_hbm.at[didx], rows, sem).wait()
      pltpu.sync_copy(rows, xs_hbm.at[pl.ds(base_x + q * CK, CK)])

  return k(dstf, ewf, x1p, permp)


def _tc_layer_in(xin, w1s, deg2):
  def body(x_ref, w_ref, d_ref, o_ref):
    dinv = lax.rsqrt(d_ref[0] + 1.0)
    h = jnp.dot(x_ref[0, 0], w_ref[0], preferred_element_type=_f32)
    o_ref[0, 0] = dinv * h

  return pl.pallas_call(
      body,
      grid=(2, 2, GR),
      in_specs=[
          pl.BlockSpec((1, 1, BR, F), lambda v, p, r: (v, p, r, 0)),
          pl.BlockSpec((1, F, F), lambda v, p, r: (v, 0, 0)),
          pl.BlockSpec((1, BR, F), lambda v, p, r: (v, r, 0)),
      ],
      out_specs=pl.BlockSpec((1, 1, BR, F), lambda v, p, r: (v, p, r, 0)),
      out_shape=jax.ShapeDtypeStruct((2, 2, NP, F), _f32),
  )(xin, w1s, deg2)


def _tc_mid(s1, h1, deg2, b1s, a_s, w2s):
  def body(s_ref, h_ref, d_ref, b_ref, a_ref, w_ref, o_ref):
    v = pl.program_id(0)
    dinv = lax.rsqrt(d_ref[0] + 1.0)
    b = jnp.where(v == 0, b_ref[0], b_ref[1])
    a = jnp.where(v == 0, a_ref[0], a_ref[1])
    z = dinv * (s_ref[0, 0] + h_ref[0, 0]) + b
    z = jnp.where(z > 0, z, a * z)
    o_ref[0, 0] = dinv * jnp.dot(z, w_ref[0], preferred_element_type=_f32)

  return pl.pallas_call(
      body,
      grid=(2, 2, GR),
      in_specs=[
          pl.BlockSpec((1, 1, BR, F), lambda v, p, r: (v, p, r, 0)),
          pl.BlockSpec((1, 1, BR, F), lambda v, p, r: (v, p, r, 0)),
          pl.BlockSpec((1, BR, F), lambda v, p, r: (v, r, 0)),
          pl.BlockSpec((2, F), lambda v, p, r: (0, 0)),
          pl.BlockSpec((2, F), lambda v, p, r: (0, 0)),
          pl.BlockSpec((1, F, F), lambda v, p, r: (v, 0, 0)),
      ],
      out_specs=pl.BlockSpec((1, 1, BR, F), lambda v, p, r: (v, p, r, 0)),
      out_shape=jax.ShapeDtypeStruct((2, 2, NP, F), _f32),
  )(s1, h1, deg2, b1s, a_s, w2s)


def _tc_out(s2, h2, deg2, b2s):
  def body(s_ref, h_ref, d_ref, b_ref, o_ref, c_ref):
    v = pl.program_id(0)
    p = pl.program_id(1)
    r = pl.program_id(2)
    dinv = lax.rsqrt(d_ref[0] + 1.0)
    b = jnp.where(v == 0, b_ref[0], b_ref[1])
    o = dinv * (s_ref[0, 0] + h_ref[0, 0]) + b
    o_ref[0, 0] = o

    @pl.when(jnp.logical_and(v == 0, jnp.logical_and(p == 0, r == 0)))
    def _():
      c_ref[...] = jnp.zeros((2, F), _f32)

    @pl.when(p == 0)
    def _():
      rows = r * BR + lax.broadcasted_iota(jnp.int32, (BR, F), 0)
      contrib = jnp.sum(jnp.where(rows < N, o, 0.0), axis=0)

      @pl.when(v == 0)
      def _():
        c_ref[0, :] += contrib

      @pl.when(v == 1)
      def _():
        c_ref[1, :] += contrib

  return pl.pallas_call(
      body,
      grid=(2, 2, GR),
      in_specs=[
          pl.BlockSpec((1, 1, BR, F), lambda v, p, r: (v, p, r, 0)),
          pl.BlockSpec((1, 1, BR, F), lambda v, p, r: (v, p, r, 0)),
          pl.BlockSpec((1, BR, F), lambda v, p, r: (v, r, 0)),
          pl.BlockSpec((2, F), lambda v, p, r: (0, 0)),
      ],
      out_specs=[
          pl.BlockSpec((1, 1, BR, F), lambda v, p, r: (v, p, r, 0)),
          pl.BlockSpec((2, F), lambda v, p, r: (0, 0)),
      ],
      out_shape=[
          jax.ShapeDtypeStruct((2, 2, NP, F), _f32),
          jax.ShapeDtypeStruct((2, F), _f32),
      ],
  )(s2, h2, deg2, b2s)


def _tc_logits(outv, csum, wd):
  def body(o_ref, c_ref, w_ref, l_ref):
    v = pl.program_id(2)
    cv = jnp.where(v == 0, c_ref[0], c_ref[1])
    cvec = jax.nn.sigmoid(cv * (1.0 / N))
    wc = jnp.sum(w_ref[...] * cvec[None, :], axis=1)
    sc = jnp.sum(o_ref[0, 0] * wc[None, :], axis=1)

    @pl.when(v == 0)
    def _():
      l_ref[0, :] = sc

    @pl.when(v == 1)
    def _():
      l_ref[1, :] = sc

  return pl.pallas_call(
      body,
      grid=(2, GR, 2),
      in_specs=[
          pl.BlockSpec((1, 1, BR, F), lambda p, r, v: (v, p, r, 0)),
          pl.BlockSpec((2, F), lambda p, r, v: (0, 0)),
          pl.BlockSpec((F, F), lambda p, r, v: (0, 0)),
      ],
      out_specs=pl.BlockSpec((2, BR), lambda p, r, v: (0, p * GR + r)),
      out_shape=jax.ShapeDtypeStruct((2, 2 * NP), _f32),
  )(outv, csum, wd)


def kernel(x0, edge_index0, edge_weight0, x1, edge_index1, edge_weight1,
           W1_0, b1_0, a_0, W2_0, b2_0, W1_1, b1_1, a_1, W2_1, b2_1, Wd):
  x0p = jnp.pad(x0, ((0, NP - N), (0, 0)))
  x1p = jnp.pad(x1, ((0, NP - N), (0, 0)))
  perm = jax.random.permutation(jax.random.key(42), N).astype(jnp.int32)
  permp = jnp.pad(perm, (0, NP - N))

  dstf = jnp.concatenate([edge_index0[1], edge_index1[1]])
  ewf = jnp.concatenate([edge_weight0, edge_weight1])
  degf, xsp = _sc_deg_shuffle(dstf, ewf, x1p, permp)
  deg2 = degf.reshape(NC, NP, F)

  xin = jnp.stack([jnp.stack([x0p, xsp]), jnp.stack([x1p, xsp])])
  w1s = jnp.stack([W1_0, W1_1])
  w2s = jnp.stack([W2_0, W2_1])
  b1s = jnp.stack([b1_0, b1_1])
  b2s = jnp.stack([b2_0, b2_1])
  a_s = jnp.stack([jnp.broadcast_to(a_0, (F,)), jnp.broadcast_to(a_1, (F,))])

  h1 = _tc_layer_in(xin, w1s, deg2)

  s1_list = []
  for v in range(2):
    src, dst, ew = (edge_index0[0], edge_index0[1], edge_weight0) if v == 0 \
        else (edge_index1[0], edge_index1[1], edge_weight1)
    s1_list.append(_sc_spmm(h1[v].reshape(NC * NP, F), src, dst, ew))
  s1 = jnp.stack([s.reshape(NC, NP, F) for s in s1_list])

  h2 = _tc_mid(s1, h1, deg2, b1s, a_s, w2s)

  s2_list = []
  for v in range(2):
    src, dst, ew = (edge_index0[0], edge_index0[1], edge_weight0) if v == 0 \
        else (edge_index1[0], edge_index1[1], edge_weight1)
    s2_list.append(_sc_spmm(h2[v].reshape(NC * NP, F), src, dst, ew))
  s2 = jnp.stack([s.reshape(NC, NP, F) for s in s2_list])

  outv, csum = _tc_out(s2, h2, deg2, b2s)
  logits_p = _tc_logits(outv, csum, Wd)
  return jnp.concatenate([logits_p[:, :N], logits_p[:, NP:NP + N]], axis=1)
